# baseline (speedup 1.0000x reference)
import functools

import jax
import jax.numpy as jnp
from jax import lax
from jax.experimental import pallas as pl
from jax.experimental.pallas import tpu as pltpu

_CAPACITY_FACTOR = 1.25
_MIN_CAPACITY = 4


def _capacity(num_tokens, num_experts):
    cap = int(_CAPACITY_FACTOR * num_tokens / num_experts)
    cap += cap % 2
    return max(cap, _MIN_CAPACITY)


def _router_fill_body(x_ref, out_ref, pk_ref, counts_ref, *, cap, blk):
    i = pl.program_id(0)

    @pl.when(i == 0)
    def _init():
        counts_ref[...] = jnp.zeros_like(counts_ref)

    x = x_ref[...]
    e = x.shape[-1]
    row = e * cap
    m = jnp.max(x, axis=-1, keepdims=True)
    ex = jnp.exp(x - m)
    s = jnp.sum(ex, axis=-1, keepdims=True)
    logits = ex / s

    e_iota = lax.broadcasted_iota(jnp.int32, (blk, e), 1)
    is_max = x == m
    top1 = jnp.min(jnp.where(is_max, e_iota, e), axis=-1, keepdims=True)
    mask = (e_iota == top1).astype(jnp.float32)

    r_iota = lax.broadcasted_iota(jnp.int32, (blk, blk), 0)
    c_iota = lax.broadcasted_iota(jnp.int32, (blk, blk), 1)
    ltri = (r_iota > c_iota).astype(jnp.float32)
    excl = jax.lax.dot_general(
        ltri, mask, (((1,), (0,)), ((), ())),
        preferred_element_type=jnp.float32)
    ranks = excl + counts_ref[...]
    counts_ref[...] = counts_ref[...] + jnp.sum(mask, axis=0, keepdims=True)

    keep = mask * (ranks < cap).astype(jnp.float32)
    kept = jnp.sum(keep, axis=-1, keepdims=True) > 0.0
    rank_tok = jnp.sum(ranks * keep, axis=-1, keepdims=True).astype(jnp.int32)
    w_tok = jnp.sum(logits * keep, axis=-1, keepdims=True)
    tgt = jnp.where(kept, top1 * cap + rank_tok, -1)

    col = lax.broadcasted_iota(jnp.int32, (blk, row), 1)
    out_ref[...] = jnp.where(col == tgt, w_tok, 0.0)

    colw = lax.broadcasted_iota(jnp.int32, (blk, row // 32), 1)
    pk_ref[...] = jnp.where(colw == (tgt >> 5), 1 << (tgt & 31), 0)


def kernel(inputs):
    n, e = inputs.shape
    cap = _capacity(n, e)
    row = e * cap
    blk = 256
    grid = n // blk
    x = inputs.astype(jnp.float32)

    out, packed = pl.pallas_call(
        functools.partial(_router_fill_body, cap=cap, blk=blk),
        grid=(grid,),
        in_specs=[pl.BlockSpec((blk, e), lambda i: (i, 0))],
        out_specs=[
            pl.BlockSpec((blk, row), lambda i: (i, 0)),
            pl.BlockSpec((blk, row // 32), lambda i: (i, 0)),
        ],
        out_shape=[
            jax.ShapeDtypeStruct((n, row), jnp.float32),
            jax.ShapeDtypeStruct((n, row // 32), jnp.int32),
        ],
        scratch_shapes=[pltpu.VMEM((1, e), jnp.float32)],
    )(x)

    bits = jnp.arange(32, dtype=jnp.int32)[None, None, :]
    sec = ((packed[:, :, None] >> bits) & 1) != 0
    return (out.reshape(n, e, cap).astype(inputs.dtype),
            sec.reshape(n, e, cap))
